# Initial kernel scaffold; baseline (speedup 1.0000x reference)
import jax
import jax.numpy as jnp
from jax.experimental import pallas as pl


def kernel(x, edge_index, Qw, Qb, Kw, Kb, Vw, Vb):
    raise NotImplementedError("write your pallas kernel here")


# speedup vs baseline: 37.3269x; 37.3269x over previous
import jax
import jax.numpy as jnp
from jax import lax
from jax.experimental import pallas as pl
from jax.experimental.pallas import tpu as pltpu
from jax.experimental.pallas import tpu_sc as plsc

_N = 10000
_E = 320000
_IN = 128
_H = 8
_D = 16
_HD = _H * _D

_NC = 2
_NS = 16
_EPC = _E // _NC
_EPW = _EPC // _NS
_CB = 80
_NCH = _EPW // _CB
_NRC = _N // _CB
_RCPS = -(-_NRC // _NS)

_BLK = 1000


def _proj_body(x_ref, qw_ref, kw_ref, vw_ref, qb_ref, kb_ref, vb_ref,
               q_ref, kv_ref):
    xb = x_ref[...]
    q_ref[...] = jnp.dot(xb, qw_ref[...], preferred_element_type=jnp.float32) + qb_ref[...]
    kv_ref[:, :_HD] = jnp.dot(xb, kw_ref[...], preferred_element_type=jnp.float32) + kb_ref[...]
    kv_ref[:, _HD:] = jnp.dot(xb, vw_ref[...], preferred_element_type=jnp.float32) + vb_ref[...]


_proj = pl.pallas_call(
    _proj_body,
    grid=(_N // _BLK,),
    in_specs=[
        pl.BlockSpec((_BLK, _IN), lambda i: (i, 0)),
        pl.BlockSpec((_IN, _HD), lambda i: (0, 0)),
        pl.BlockSpec((_IN, _HD), lambda i: (0, 0)),
        pl.BlockSpec((_IN, _HD), lambda i: (0, 0)),
        pl.BlockSpec((1, _HD), lambda i: (0, 0)),
        pl.BlockSpec((1, _HD), lambda i: (0, 0)),
        pl.BlockSpec((1, _HD), lambda i: (0, 0)),
    ],
    out_specs=[pl.BlockSpec((_BLK, _HD), lambda i: (i, 0)),
               pl.BlockSpec((_BLK, 2 * _HD), lambda i: (i, 0))],
    out_shape=[jax.ShapeDtypeStruct((_N, _HD), jnp.float32),
               jax.ShapeDtypeStruct((_N, 2 * _HD), jnp.float32)],
)


def _sum_body(p_ref, o_ref):
    o_ref[...] = p_ref[0] + p_ref[1]


_sum2 = pl.pallas_call(
    _sum_body,
    grid=(_N // _BLK,),
    in_specs=[pl.BlockSpec((_NC, _BLK, _HD), lambda i: (0, i, 0))],
    out_specs=pl.BlockSpec((_BLK, _HD), lambda i: (i, 0)),
    out_shape=jax.ShapeDtypeStruct((_N, _HD), jnp.float32),
)


def _edge_body(qh, kvh, src, dst, out,
               wv, srcv, dstv, kvrows, qrows, msg, sem):
    cid = lax.axis_index("c")
    sid = lax.axis_index("s")
    zero16 = jnp.zeros((_D,), jnp.float32)
    iot = lax.iota(jnp.int32, _D)

    def _zmsg(i, carry):
        r = i // (_HD // _D)
        c = i % (_HD // _D)
        msg[r, pl.ds(c * _D, _D)] = zero16
        return carry

    lax.fori_loop(0, _CB * (_HD // _D), _zmsg, 0)

    def _zchunk(j, carry):
        ridx = sid + j * _NS

        @pl.when(ridx < _NRC)
        def _():
            pltpu.sync_copy(msg, wv.at[pl.ds(ridx * _CB, _CB)])

        return carry

    lax.fori_loop(0, _RCPS, _zchunk, 0)
    plsc.subcore_barrier()

    ebase = cid * _EPC + sid * _EPW

    def _chunk(g, carry):
        base = ebase + g * _CB
        pltpu.sync_copy(src.at[pl.ds(base, _CB)], srcv)
        pltpu.sync_copy(dst.at[pl.ds(base, _CB)], dstv)
        pltpu.async_copy(kvh.at[srcv], kvrows, sem).wait()
        pltpu.async_copy(qh.at[dstv], qrows, sem).wait()

        def _pair(p, c2):
            e0 = p * 2
            svec = jnp.zeros((_D,), jnp.float32)
            for j in range(2 * _H):
                e = e0 + (j // _H)
                h = j % _H
                kc = kvrows[e, pl.ds(h * _D, _D)]
                qc = qrows[e, pl.ds(h * _D, _D)]
                s = jnp.sum(kc * qc)
                svec = jnp.where(iot == j, s, svec)
            svec = jnp.exp(lax.clamp(-5.0, svec * 0.25, 5.0))
            for j in range(2 * _H):
                e = e0 + (j // _H)
                h = j % _H
                vc = kvrows[e, pl.ds(_HD + h * _D, _D)]
                msg[e, pl.ds(h * _D, _D)] = vc * svec[j]
            return c2

        lax.fori_loop(0, _CB // 2, _pair, 0)
        pltpu.sync_copy(msg, wv.at[dstv], add=True)
        return carry

    lax.fori_loop(0, _NCH, _chunk, 0)

    plsc.subcore_barrier()

    def _dchunk(j, carry):
        ridx = sid + j * _NS

        @pl.when(ridx < _NRC)
        def _():
            pltpu.sync_copy(wv.at[pl.ds(ridx * _CB, _CB)],
                            out.at[cid, pl.ds(ridx * _CB, _CB)])

        return carry

    lax.fori_loop(0, _RCPS, _dchunk, 0)


_edge_kernel = pl.kernel(
    _edge_body,
    out_type=jax.ShapeDtypeStruct((_NC, _N, _HD), jnp.float32),
    mesh=plsc.VectorSubcoreMesh(core_axis_name="c", subcore_axis_name="s"),
    compiler_params=pltpu.CompilerParams(needs_layout_passes=False),
    scratch_types=[
        pltpu.VMEM_SHARED((_N, _HD), jnp.float32),
        pltpu.VMEM((_CB,), jnp.int32),
        pltpu.VMEM((_CB,), jnp.int32),
        pltpu.VMEM((_CB, 2 * _HD), jnp.float32),
        pltpu.VMEM((_CB, _HD), jnp.float32),
        pltpu.VMEM((_CB, _HD), jnp.float32),
        pltpu.SemaphoreType.DMA,
    ],
)


def kernel(x, edge_index, Qw, Qb, Kw, Kb, Vw, Vb):
    q, kv = _proj(x, Qw, Kw, Vw,
                  Qb.reshape(1, _HD), Kb.reshape(1, _HD), Vb.reshape(1, _HD))
    src = edge_index[0].astype(jnp.int32)
    dst = edge_index[1].astype(jnp.int32)
    parts = _edge_kernel(q, kv, src, dst)
    wv = _sum2(parts)
    return wv.reshape(_N, _H, _D)
